# baseline (speedup 1.0000x reference)
import functools

import jax
import jax.numpy as jnp
from jax import lax
from jax.experimental import pallas as pl
from jax.experimental.pallas import tpu as pltpu
from jax.experimental.pallas import tpu_sc as plsc

GAP_VAL = 0.05
ROWS, COLS = 64, 32768
LANES = 16
NUM_CORES, NUM_SUBCORES = 2, 16
NUM_WORKERS = NUM_CORES * NUM_SUBCORES
ROWS_PER_WORKER = ROWS // NUM_WORKERS
RG, CT, RSUB, CSUB = 8, 256, 8, 128
BLOCKS = 16
BLOCK_TILES = CT // BLOCKS
TILE_CHUNKS = CSUB // LANES

_mesh = plsc.VectorSubcoreMesh(core_axis_name="c", subcore_axis_name="s")


@functools.partial(
    pl.kernel,
    out_type=jax.ShapeDtypeStruct((RG, CT, RSUB, CSUB), jnp.float32),
    mesh=_mesh,
    scratch_types=[
        pltpu.VMEM((ROWS_PER_WORKER, CT, CSUB), jnp.float32),
        pltpu.VMEM((ROWS_PER_WORKER, BLOCKS, LANES), jnp.float32),
        pltpu.SemaphoreType.DMA,
        pltpu.SemaphoreType.DMA,
        pltpu.SemaphoreType.DMA,
        pltpu.SemaphoreType.DMA,
        pltpu.SemaphoreType.DMA,
        pltpu.SemaphoreType.DMA,
    ],
    compiler_params=pltpu.CompilerParams(needs_layout_passes=False),
)
def _renetwork(act_hbm, out_hbm, buf, bmref,
               sem0a, sem0b, sem1a, sem1b, semo0, semo1):
    wid = lax.axis_index("s") * NUM_CORES + lax.axis_index("c")
    g = wid // 4
    rr0 = 2 * (wid % 4)
    in_sems = ((sem0a, sem0b), (sem1a, sem1b))
    out_sems = (semo0, semo1)
    HT = CT // 2
    in_copies = [
        [pltpu.async_copy(act_hbm.at[g, pl.ds(h * HT, HT), rr0 + k, :],
                          buf.at[k, pl.ds(h * HT, HT), :], in_sems[k][h])
         for h in range(2)]
        for k in range(ROWS_PER_WORKER)
    ]
    out_copies = []
    leads = []
    for k in range(ROWS_PER_WORKER):
        def p1_body(b, rowmax, k=k):
            def tile_body(t, m):
                for j in range(TILE_CHUNKS):
                    m = jnp.maximum(
                        m,
                        buf[k, b * BLOCK_TILES + t, pl.ds(j * LANES, LANES)])
                return m

            bm = lax.fori_loop(0, BLOCK_TILES, tile_body,
                               jnp.full((LANES,), -jnp.inf, jnp.float32))
            bmref[k, b] = bm
            return jnp.maximum(rowmax, bm)

        m = jnp.full((LANES,), -jnp.inf, jnp.float32)
        for h in range(2):
            in_copies[k][h].wait()
            m = lax.fori_loop(h * BLOCKS // 2, (h + 1) * BLOCKS // 2,
                              p1_body, m)
        for q in (1, 2, 4, 8):
            idx = lax.iota(jnp.int32, LANES) ^ q
            m = jnp.maximum(m, m.at[idx].get(mode="promise_in_bounds"))
        leads.append(m)

        out_copies.append(
            pltpu.async_copy(buf.at[k], out_hbm.at[g, :, rr0 + k, :],
                             out_sems[k]))

        def fix_body(b, carry, k=k, lead=m):
            near = (lead - bmref[k, b]) < GAP_VAL
            may_hit = plsc.all_reduce_population_count(near)[0] > 0

            @pl.when(may_hit)
            def _():
                def tile_body(t, c):
                    for j in range(TILE_CHUNKS):
                        v = buf[k, b * BLOCK_TILES + t, pl.ds(j * LANES, LANES)]
                        interference = lead - v
                        hit = (interference > 0.0) & (interference < GAP_VAL)
                        buf[k, b * BLOCK_TILES + t, pl.ds(j * LANES, LANES)] = (
                            jnp.where(hit, v - 150.0, v))
                    return c

                lax.fori_loop(0, BLOCK_TILES, tile_body, 0)

            return carry

        lax.fori_loop(0, BLOCKS, fix_body, 0)

    for k in range(ROWS_PER_WORKER):
        out_copies[k].wait()

        def rewrite_body(b, carry, k=k, lead=leads[k]):
            near = (lead - bmref[k, b]) < GAP_VAL
            may_hit = plsc.all_reduce_population_count(near)[0] > 0

            @pl.when(may_hit)
            def _():
                pltpu.sync_copy(
                    buf.at[k, pl.ds(b * BLOCK_TILES, BLOCK_TILES), :],
                    out_hbm.at[g, pl.ds(b * BLOCK_TILES, BLOCK_TILES),
                               rr0 + k, :])

            return carry

        lax.fori_loop(0, BLOCKS, rewrite_body, 0)


def kernel(activations):
    tiled_view = activations.reshape(RG, RSUB, CT, CSUB).transpose(0, 2, 1, 3)
    out_view = _renetwork(tiled_view)
    return out_view.transpose(0, 2, 1, 3).reshape(ROWS, COLS)


# speedup vs baseline: 1.0106x; 1.0106x over previous
import functools

import jax
import jax.numpy as jnp
from jax import lax
from jax.experimental import pallas as pl
from jax.experimental.pallas import tpu as pltpu
from jax.experimental.pallas import tpu_sc as plsc

GAP_VAL = 0.05
ROWS, COLS = 64, 32768
LANES = 16
NUM_CORES, NUM_SUBCORES = 2, 16
NUM_WORKERS = NUM_CORES * NUM_SUBCORES
ROWS_PER_WORKER = ROWS // NUM_WORKERS
RG, CT, RSUB, CSUB = 8, 256, 8, 128
BLOCKS = 16
BLOCK_TILES = CT // BLOCKS
TILE_CHUNKS = CSUB // LANES

_mesh = plsc.VectorSubcoreMesh(core_axis_name="c", subcore_axis_name="s")


@functools.partial(
    pl.kernel,
    out_type=jax.ShapeDtypeStruct((RG, CT, RSUB, CSUB), jnp.float32),
    mesh=_mesh,
    scratch_types=[
        pltpu.VMEM((ROWS_PER_WORKER, CT, CSUB), jnp.float32),
        pltpu.VMEM((ROWS_PER_WORKER, BLOCKS, LANES), jnp.float32),
        pltpu.SemaphoreType.DMA,
        pltpu.SemaphoreType.DMA,
        pltpu.SemaphoreType.DMA,
        pltpu.SemaphoreType.DMA,
    ],
    compiler_params=pltpu.CompilerParams(needs_layout_passes=False),
)
def _renetwork(act_hbm, out_hbm, buf, bmref, sem0, sem1, semo0, semo1):
    wid = lax.axis_index("s") * NUM_CORES + lax.axis_index("c")
    g = wid // 4
    rr0 = 2 * (wid % 4)
    in_sems = (sem0, sem1)
    out_sems = (semo0, semo1)
    in_copies = [
        pltpu.async_copy(act_hbm.at[g, :, rr0 + k, :], buf.at[k], in_sems[k])
        for k in range(ROWS_PER_WORKER)
    ]
    out_copies = []
    leads = []
    for k in range(ROWS_PER_WORKER):
        in_copies[k].wait()

        def p1_body(b, rowmax, k=k):
            def tile_body(t, m):
                for j in range(TILE_CHUNKS):
                    m = jnp.maximum(
                        m,
                        buf[k, b * BLOCK_TILES + t, pl.ds(j * LANES, LANES)])
                return m

            bm = lax.fori_loop(0, BLOCK_TILES, tile_body,
                               jnp.full((LANES,), -jnp.inf, jnp.float32))
            bmref[k, b] = bm
            return jnp.maximum(rowmax, bm)

        m = lax.fori_loop(0, BLOCKS, p1_body,
                          jnp.full((LANES,), -jnp.inf, jnp.float32))
        for q in (1, 2, 4, 8):
            idx = lax.iota(jnp.int32, LANES) ^ q
            m = jnp.maximum(m, m.at[idx].get(mode="promise_in_bounds"))
        leads.append(m)

        out_copies.append(
            pltpu.async_copy(buf.at[k], out_hbm.at[g, :, rr0 + k, :],
                             out_sems[k]))

        def fix_body(b, carry, k=k, lead=m):
            near = (lead - bmref[k, b]) < GAP_VAL
            may_hit = plsc.all_reduce_population_count(near)[0] > 0

            @pl.when(may_hit)
            def _():
                def tile_body(t, c):
                    for j in range(TILE_CHUNKS):
                        v = buf[k, b * BLOCK_TILES + t, pl.ds(j * LANES, LANES)]
                        interference = lead - v
                        hit = (interference > 0.0) & (interference < GAP_VAL)
                        buf[k, b * BLOCK_TILES + t, pl.ds(j * LANES, LANES)] = (
                            jnp.where(hit, v - 150.0, v))
                    return c

                lax.fori_loop(0, BLOCK_TILES, tile_body, 0)

            return carry

        lax.fori_loop(0, BLOCKS, fix_body, 0)

    for k in range(ROWS_PER_WORKER):
        out_copies[k].wait()

        def rewrite_body(b, carry, k=k, lead=leads[k]):
            near = (lead - bmref[k, b]) < GAP_VAL
            may_hit = plsc.all_reduce_population_count(near)[0] > 0

            @pl.when(may_hit)
            def _():
                pltpu.sync_copy(
                    buf.at[k, pl.ds(b * BLOCK_TILES, BLOCK_TILES), :],
                    out_hbm.at[g, pl.ds(b * BLOCK_TILES, BLOCK_TILES),
                               rr0 + k, :])

            return carry

        lax.fori_loop(0, BLOCKS, rewrite_body, 0)


def kernel(activations):
    tiled_view = activations.reshape(RG, RSUB, CT, CSUB).transpose(0, 2, 1, 3)
    out_view = _renetwork(tiled_view)
    return out_view.transpose(0, 2, 1, 3).reshape(ROWS, COLS)
